# Initial kernel scaffold; baseline (speedup 1.0000x reference)
import jax
import jax.numpy as jnp
from jax.experimental import pallas as pl


def kernel(x_user, W_user, b_user, item_emb, W_rel, W_root, bias, edge_index, edge_type):
    raise NotImplementedError("write your pallas kernel here")


# speedup vs baseline: 1.4422x; 1.4422x over previous
import functools
import jax
import jax.numpy as jnp
from jax.experimental import pallas as pl
from jax.experimental.pallas import tpu as pltpu

N_USER_K = 30000
N_ITEM_K = 20000
D_IN_K = 128
H_K = 64
R_K = 4
L_K = 3


def _user_mm_body(x_ref, w_ref, b_ref, o_ref):
    o_ref[...] = jnp.dot(x_ref[...], w_ref[...],
                         preferred_element_type=jnp.float32) + b_ref[...]


def _user_matmul(x_user, W_user, b_user):
    M, K = x_user.shape
    H = W_user.shape[1]
    BM = 1000
    return pl.pallas_call(
        _user_mm_body,
        grid=(M // BM,),
        in_specs=[
            pl.BlockSpec((BM, K), lambda i: (i, 0)),
            pl.BlockSpec((K, H), lambda i: (0, 0)),
            pl.BlockSpec((1, H), lambda i: (0, 0)),
        ],
        out_specs=pl.BlockSpec((BM, H), lambda i: (i, 0)),
        out_shape=jax.ShapeDtypeStruct((M, H), jnp.float32),
    )(x_user, W_user, b_user.reshape(1, H))


def _layer_mm_body(relu, x_ref, wroot_ref, wrel_ref, bias_ref, root_ref, y_ref):
    x = x_ref[...]
    if relu:
        x = jnp.maximum(x, 0.0)
    root_ref[...] = jnp.dot(x, wroot_ref[...],
                            preferred_element_type=jnp.float32) + bias_ref[...]
    for r in range(R_K):
        y_ref[r] = jnp.dot(x, wrel_ref[r],
                           preferred_element_type=jnp.float32)


def _layer_matmul(x, W_root_l, W_rel_l, bias_l, relu):
    N, H = x.shape
    BM = 1000
    root, y = pl.pallas_call(
        functools.partial(_layer_mm_body, relu),
        grid=(N // BM,),
        in_specs=[
            pl.BlockSpec((BM, H), lambda i: (i, 0)),
            pl.BlockSpec((H, H), lambda i: (0, 0)),
            pl.BlockSpec((R_K, H, H), lambda i: (0, 0, 0)),
            pl.BlockSpec((1, H), lambda i: (0, 0)),
        ],
        out_specs=[
            pl.BlockSpec((BM, H), lambda i: (i, 0)),
            pl.BlockSpec((R_K, BM, H), lambda i: (0, i, 0)),
        ],
        out_shape=[
            jax.ShapeDtypeStruct((N, H), jnp.float32),
            jax.ShapeDtypeStruct((R_K, N, H), jnp.float32),
        ],
    )(x, W_root_l, W_rel_l, bias_l.reshape(1, H))
    return root, y


def kernel(x_user, W_user, b_user, item_emb, W_rel, W_root, bias, edge_index, edge_type):
    N = N_USER_K + N_ITEM_K
    h_user = _user_matmul(x_user, W_user, b_user)
    x = jnp.concatenate([h_user, item_emb], axis=0)
    src = edge_index[0]
    dst = edge_index[1]
    onehot_idx = edge_type * N + dst
    c = jax.ops.segment_sum(jnp.ones((src.shape[0],), jnp.float32),
                            onehot_idx, num_segments=R_K * N)
    inv = 1.0 / jnp.clip(c, 1.0)
    s_e = inv[onehot_idx]
    gidx = edge_type * N + src
    for l in range(L_K):
        root, y = _layer_matmul(x, W_root[l], W_rel[l], bias[l], relu=(l > 0))
        yflat = y.reshape(R_K * N, H_K)
        msg = yflat[gidx] * s_e[:, None]
        agg = jax.ops.segment_sum(msg, dst, num_segments=N)
        x = root + agg
    return x


# speedup vs baseline: 5.0878x; 3.5278x over previous
import functools
import jax
import jax.numpy as jnp
from jax import lax
from jax.experimental import pallas as pl
from jax.experimental.pallas import tpu as pltpu
from jax.experimental.pallas import tpu_sc as plsc

N_K = 50000
H_K = 64
R_K = 4
L_K = 3
E_K = 800000
E_PAD = 802816
PAD_DST = 50001

N_HALF = 25000
TRASH_ROW = 25088
ACC_ROWS = 25096

CT_HALF = 102400
CT_TRASH = 100096
INV_SZ = 2 * CT_HALF

EPT_P1 = 50176
EPW_P2 = 25088

_MESH = plsc.VectorSubcoreMesh(core_axis_name="c", subcore_axis_name="s")
_SC_PARAMS = pltpu.CompilerParams(needs_layout_passes=False,
                                  use_tc_tiling_on_sc=False)


def _user_mm_body(x_ref, w_ref, b_ref, o_ref):
    o_ref[...] = jnp.dot(x_ref[...], w_ref[...],
                         preferred_element_type=jnp.float32) + b_ref[...]


def _user_matmul(x_user, W_user, b_user):
    M, K = x_user.shape
    H = W_user.shape[1]
    BM = 1000
    return pl.pallas_call(
        _user_mm_body,
        grid=(M // BM,),
        in_specs=[
            pl.BlockSpec((BM, K), lambda i: (i, 0)),
            pl.BlockSpec((K, H), lambda i: (0, 0)),
            pl.BlockSpec((1, H), lambda i: (0, 0)),
        ],
        out_specs=pl.BlockSpec((BM, H), lambda i: (i, 0)),
        out_shape=jax.ShapeDtypeStruct((M, H), jnp.float32),
    )(x_user, W_user, b_user.reshape(1, H))


def _layer_mm_body(relu, x_ref, wroot_ref, wrel_ref, bias_ref, root_ref, y_ref):
    x = x_ref[...]
    if relu:
        x = jnp.maximum(x, 0.0)
    root_ref[...] = jnp.dot(x, wroot_ref[...],
                            preferred_element_type=jnp.float32) + bias_ref[...]
    for r in range(R_K):
        y_ref[r] = jnp.dot(x, wrel_ref[r],
                           preferred_element_type=jnp.float32)


def _layer_matmul(x, W_root_l, W_rel_l, bias_l, relu):
    N, H = x.shape
    BM = 1000
    return pl.pallas_call(
        functools.partial(_layer_mm_body, relu),
        grid=(N // BM,),
        in_specs=[
            pl.BlockSpec((BM, H), lambda i: (i, 0)),
            pl.BlockSpec((H, H), lambda i: (0, 0)),
            pl.BlockSpec((R_K, H, H), lambda i: (0, 0, 0)),
            pl.BlockSpec((1, H), lambda i: (0, 0)),
        ],
        out_specs=[
            pl.BlockSpec((BM, H), lambda i: (i, 0)),
            pl.BlockSpec((R_K, BM, H), lambda i: (0, i, 0)),
        ],
        out_shape=[
            jax.ShapeDtypeStruct((N, H), jnp.float32),
            jax.ShapeDtypeStruct((R_K, N, H), jnp.float32),
        ],
    )(x, W_root_l, W_rel_l, bias_l.reshape(1, H))


def _prep1_body(dst_hbm, t_hbm, inv_hbm, counts_sh, zbuf, ones_b, dbuf, tbuf,
                c0, c1, c2, c3):
    cid = lax.axis_index("c")
    sid = lax.axis_index("s")
    crefs = [c0, c1, c2, c3]

    def zero_body(i, _):
        zbuf[pl.ds(i * 16, 16)] = jnp.zeros((16,), jnp.float32)
        return 0
    lax.fori_loop(0, 400, zero_body, 0)

    def ones_body(i, _):
        ones_b[pl.ds(i * 16, 16)] = jnp.ones((16,), jnp.float32)
        return 0
    lax.fori_loop(0, 8, ones_body, 0)

    pltpu.sync_copy(zbuf.at[pl.ds(0, 6400)],
                    counts_sh.at[pl.ds(sid * 6400, 6400)])
    plsc.subcore_barrier()

    nbase = cid * N_HALF

    def super_body(sup, _):
        base = sid * EPT_P1 + sup * 512
        pltpu.sync_copy(dst_hbm.at[pl.ds(base, 512)], dbuf)
        pltpu.sync_copy(t_hbm.at[pl.ds(base, 512)], tbuf)
        for j in range(4):
            cref = crefs[j]

            def cbody(k, _c):
                off = j * 128 + k * 16
                d16 = dbuf[pl.ds(off, 16)]
                t16 = tbuf[pl.ds(off, 16)]
                loc = d16 - nbase
                ok = (loc >= 0) & (loc < N_HALF)
                cref[pl.ds(k * 16, 16)] = jnp.where(ok, loc * R_K + t16,
                                                    CT_TRASH)
                return 0
            lax.fori_loop(0, 8, cbody, 0)
            pltpu.sync_copy(ones_b, counts_sh.at[cref], add=True)
        return 0
    lax.fori_loop(0, EPT_P1 // 512, super_body, 0)

    plsc.subcore_barrier()
    pltpu.sync_copy(counts_sh.at[pl.ds(sid * 6400, 6400)],
                    zbuf.at[pl.ds(0, 6400)])

    def inv_body(i, _):
        v = zbuf[pl.ds(i * 16, 16)]
        zbuf[pl.ds(i * 16, 16)] = 1.0 / jnp.maximum(v, 1.0)
        return 0
    lax.fori_loop(0, 400, inv_body, 0)
    pltpu.sync_copy(zbuf.at[pl.ds(0, 6400)],
                    inv_hbm.at[pl.ds(cid * CT_HALF + sid * 6400, 6400)])


@functools.partial(
    pl.kernel, mesh=_MESH, compiler_params=_SC_PARAMS,
    out_type=jax.ShapeDtypeStruct((INV_SZ,), jnp.float32),
    scratch_types=[
        pltpu.VMEM_SHARED((CT_HALF,), jnp.float32),
        pltpu.VMEM((6400,), jnp.float32),
        pltpu.VMEM((128,), jnp.float32),
        pltpu.VMEM((512,), jnp.int32),
        pltpu.VMEM((512,), jnp.int32),
        pltpu.VMEM((128,), jnp.int32),
        pltpu.VMEM((128,), jnp.int32),
        pltpu.VMEM((128,), jnp.int32),
        pltpu.VMEM((128,), jnp.int32),
    ],
)
def _sc_prep1(dst_hbm, t_hbm, inv_hbm, counts_sh, zbuf, ones_b, dbuf, tbuf,
              c0, c1, c2, c3):
    _prep1_body(dst_hbm, t_hbm, inv_hbm, counts_sh, zbuf, ones_b, dbuf, tbuf,
                c0, c1, c2, c3)


@functools.partial(
    pl.kernel, mesh=_MESH, compiler_params=_SC_PARAMS,
    out_type=[
        jax.ShapeDtypeStruct((E_PAD,), jnp.int32),
        jax.ShapeDtypeStruct((E_PAD,), jnp.float32),
    ],
    scratch_types=[
        pltpu.VMEM((512,), jnp.int32),
        pltpu.VMEM((512,), jnp.int32),
        pltpu.VMEM((512,), jnp.int32),
        pltpu.VMEM((512,), jnp.int32),
        pltpu.VMEM((512,), jnp.int32),
        pltpu.VMEM((512,), jnp.float32),
        pltpu.SemaphoreType.DMA,
    ],
)
def _sc_prep2(src_hbm, dst_hbm, t_hbm, inv_hbm, gidx_hbm, s_hbm,
              bsrc, bdst, bt, gflat, cflat, sflat, sem):
    cid = lax.axis_index("c")
    sid = lax.axis_index("s")
    wid = cid * 16 + sid

    def chunk_body(ch, _):
        base = wid * EPW_P2 + ch * 512
        pltpu.sync_copy(src_hbm.at[pl.ds(base, 512)], bsrc)
        pltpu.sync_copy(dst_hbm.at[pl.ds(base, 512)], bdst)
        pltpu.sync_copy(t_hbm.at[pl.ds(base, 512)], bt)

        def cbody(i, _c):
            sl = pl.ds(i * 16, 16)
            s16 = bsrc[sl]
            d16 = bdst[sl]
            t16 = bt[sl]
            gflat[sl] = t16 * N_K + s16
            upper = d16 >= N_HALF
            loc = d16 - jnp.where(upper, N_HALF, 0)
            cflat[sl] = jnp.where(upper, CT_HALF, 0) + loc * R_K + t16
            return 0
        lax.fori_loop(0, 32, cbody, 0)

        pltpu.sync_copy(gflat, gidx_hbm.at[pl.ds(base, 512)])
        handles = [
            pltpu.async_copy(inv_hbm.at[cflat.at[pl.ds(j * 128, 128)]],
                             sflat.at[pl.ds(j * 128, 128)], sem)
            for j in range(4)
        ]
        for h in handles:
            h.wait()
        pltpu.sync_copy(sflat, s_hbm.at[pl.ds(base, 512)])
        return 0
    lax.fori_loop(0, EPW_P2 // 512, chunk_body, 0)


_IOTA16 = None


@functools.partial(
    pl.kernel, mesh=_MESH, compiler_params=_SC_PARAMS,
    out_type=jax.ShapeDtypeStruct((N_K, H_K), jnp.float32),
    scratch_types=[
        pltpu.VMEM_SHARED((ACC_ROWS, H_K), jnp.float32),
        pltpu.VMEM((256,), jnp.int32),
        pltpu.VMEM((256,), jnp.float32),
        pltpu.VMEM((256,), jnp.int32),
        pltpu.VMEM((128,), jnp.int32),
        pltpu.VMEM((128,), jnp.int32),
        pltpu.VMEM((256, H_K), jnp.float32),
        pltpu.SemaphoreType.DMA,
    ],
)
def _sc_layer(root_hbm, y_hbm, gidx_hbm, s_hbm, dst_hbm, out_hbm,
              acc_sh, gflat, sflat, dbuf,
              d0, d1, rows, sem):
    cid = lax.axis_index("c")
    sid = lax.axis_index("s")
    drefs = [d0, d1]
    nbase = cid * N_HALF
    r0 = sid * 1568

    @pl.when(sid < 15)
    def _init_main():
        pltpu.sync_copy(root_hbm.at[pl.ds(nbase + r0, 1568)],
                        acc_sh.at[pl.ds(r0, 1568)])

    @pl.when(sid == 15)
    def _init_tail():
        pltpu.sync_copy(root_hbm.at[pl.ds(nbase + 23520, 1480)],
                        acc_sh.at[pl.ds(23520, 1480)])

    plsc.subcore_barrier()

    iota16 = lax.iota(jnp.int32, 16)

    def super_body(sup, _):
        eb = sid * EPT_P1 + sup * 256
        pltpu.sync_copy(gidx_hbm.at[pl.ds(eb, 256)], gflat)
        pltpu.sync_copy(s_hbm.at[pl.ds(eb, 256)], sflat)
        pltpu.sync_copy(dst_hbm.at[pl.ds(eb, 256)], dbuf)

        for j in range(2):
            dref = drefs[j]

            def dbody(k, _c):
                d16 = dbuf[pl.ds(j * 128 + k * 16, 16)]
                loc = d16 - nbase
                ok = (loc >= 0) & (loc < N_HALF)
                dref[pl.ds(k * 16, 16)] = jnp.where(ok, loc, TRASH_ROW)
                return 0
            lax.fori_loop(0, 8, dbody, 0)

        handles = [
            pltpu.async_copy(y_hbm.at[gflat.at[pl.ds(j * 128, 128)]],
                             rows.at[pl.ds(j * 128, 128)], sem)
            for j in range(2)
        ]
        for j in range(2):
            handles[j].wait()

            def scale_body(i, _c):
                e0 = j * 128 + i * 16
                for k in range(16):
                    row = e0 + k
                    ridx = jnp.full((16,), row, jnp.int32)
                    sk = plsc.load_gather(sflat, [ridx])
                    for p in range(4):
                        cidx = iota16 + (p * 16)
                        v = plsc.load_gather(rows, [ridx, cidx])
                        plsc.store_scatter(rows, [ridx, cidx], v * sk)
                return 0
            lax.fori_loop(0, 8, scale_body, 0)
            pltpu.sync_copy(rows.at[pl.ds(j * 128, 128)],
                            acc_sh.at[drefs[j]], add=True)
        return 0
    lax.fori_loop(0, EPT_P1 // 256, super_body, 0)

    plsc.subcore_barrier()

    @pl.when(sid < 15)
    def _out_main():
        pltpu.sync_copy(acc_sh.at[pl.ds(r0, 1568)],
                        out_hbm.at[pl.ds(nbase + r0, 1568)])

    @pl.when(sid == 15)
    def _out_tail():
        pltpu.sync_copy(acc_sh.at[pl.ds(23520, 1480)],
                        out_hbm.at[pl.ds(nbase + 23520, 1480)])


def kernel(x_user, W_user, b_user, item_emb, W_rel, W_root, bias, edge_index, edge_type):
    h_user = _user_matmul(x_user, W_user, b_user)
    x = jnp.concatenate([h_user, item_emb], axis=0)

    src = edge_index[0]
    dst = edge_index[1]
    pad = E_PAD - E_K
    src_p = jnp.concatenate([src, jnp.zeros((pad,), jnp.int32)])
    dst_p = jnp.concatenate([dst, jnp.full((pad,), PAD_DST, jnp.int32)])
    t_p = jnp.concatenate([edge_type, jnp.zeros((pad,), jnp.int32)])

    inv = _sc_prep1(dst_p, t_p)
    gidx, s_e = _sc_prep2(src_p, dst_p, t_p, inv)

    for l in range(L_K):
        root, y = _layer_matmul(x, W_root[l], W_rel[l], bias[l], relu=(l > 0))
        x = _sc_layer(root, y.reshape(R_K * N_K, H_K), gidx, s_e, dst_p)
    return x


# speedup vs baseline: 5.5877x; 1.0983x over previous
import functools
import jax
import jax.numpy as jnp
from jax import lax
from jax.experimental import pallas as pl
from jax.experimental.pallas import tpu as pltpu
from jax.experimental.pallas import tpu_sc as plsc

N_K = 50000
H_K = 64
R_K = 4
L_K = 3
E_K = 800000
E_PAD = 802816
PAD_DST = 50001

N_HALF = 25000
TRASH_ROW = 25000
ACC_ROWS = 25008
CH = 112
CPB = 4
EB = CH * H_K * 4

CT_HALF = 102400
CT_TRASH = 100096
INV_SZ = 2 * CT_HALF

EPT_P1 = 50176
EPW_P2 = 25088

_MESH = plsc.VectorSubcoreMesh(core_axis_name="c", subcore_axis_name="s")
_SC_PARAMS = pltpu.CompilerParams(needs_layout_passes=False,
                                  use_tc_tiling_on_sc=False)


def _user_mm_body(x_ref, w_ref, b_ref, o_ref):
    o_ref[...] = jnp.dot(x_ref[...], w_ref[...],
                         preferred_element_type=jnp.float32) + b_ref[...]


def _user_matmul(x_user, W_user, b_user):
    M, K = x_user.shape
    H = W_user.shape[1]
    BM = 1000
    return pl.pallas_call(
        _user_mm_body,
        grid=(M // BM,),
        in_specs=[
            pl.BlockSpec((BM, K), lambda i: (i, 0)),
            pl.BlockSpec((K, H), lambda i: (0, 0)),
            pl.BlockSpec((1, H), lambda i: (0, 0)),
        ],
        out_specs=pl.BlockSpec((BM, H), lambda i: (i, 0)),
        out_shape=jax.ShapeDtypeStruct((M, H), jnp.float32),
    )(x_user, W_user, b_user.reshape(1, H))


def _layer_mm_body(relu, x_ref, wroot_ref, wrel_ref, bias_ref, root_ref, y_ref):
    x = x_ref[...]
    if relu:
        x = jnp.maximum(x, 0.0)
    root_ref[...] = jnp.dot(x, wroot_ref[...],
                            preferred_element_type=jnp.float32) + bias_ref[...]
    for r in range(R_K):
        y_ref[r] = jnp.dot(x, wrel_ref[r],
                           preferred_element_type=jnp.float32)


def _layer_matmul(x, W_root_l, W_rel_l, bias_l, relu):
    N, H = x.shape
    BM = 1000
    return pl.pallas_call(
        functools.partial(_layer_mm_body, relu),
        grid=(N // BM,),
        in_specs=[
            pl.BlockSpec((BM, H), lambda i: (i, 0)),
            pl.BlockSpec((H, H), lambda i: (0, 0)),
            pl.BlockSpec((R_K, H, H), lambda i: (0, 0, 0)),
            pl.BlockSpec((1, H), lambda i: (0, 0)),
        ],
        out_specs=[
            pl.BlockSpec((BM, H), lambda i: (i, 0)),
            pl.BlockSpec((R_K, BM, H), lambda i: (0, i, 0)),
        ],
        out_shape=[
            jax.ShapeDtypeStruct((N, H), jnp.float32),
            jax.ShapeDtypeStruct((R_K, N, H), jnp.float32),
        ],
    )(x, W_root_l, W_rel_l, bias_l.reshape(1, H))


def _prep1_body(dst_hbm, t_hbm, inv_hbm, counts_sh, zbuf, ones_b, dbuf, tbuf,
                c0, c1, c2, c3):
    cid = lax.axis_index("c")
    sid = lax.axis_index("s")
    crefs = [c0, c1, c2, c3]

    def zero_body(i, _):
        zbuf[pl.ds(i * 16, 16)] = jnp.zeros((16,), jnp.float32)
        return 0
    lax.fori_loop(0, 400, zero_body, 0)

    def ones_body(i, _):
        ones_b[pl.ds(i * 16, 16)] = jnp.ones((16,), jnp.float32)
        return 0
    lax.fori_loop(0, 8, ones_body, 0)

    pltpu.sync_copy(zbuf.at[pl.ds(0, 6400)],
                    counts_sh.at[pl.ds(sid * 6400, 6400)])
    plsc.subcore_barrier()

    nbase = cid * N_HALF

    def super_body(sup, _):
        base = sid * EPT_P1 + sup * 512
        pltpu.sync_copy(dst_hbm.at[pl.ds(base, 512)], dbuf)
        pltpu.sync_copy(t_hbm.at[pl.ds(base, 512)], tbuf)
        for j in range(4):
            cref = crefs[j]

            def cbody(k, _c):
                off = j * 128 + k * 16
                d16 = dbuf[pl.ds(off, 16)]
                t16 = tbuf[pl.ds(off, 16)]
                loc = d16 - nbase
                ok = (loc >= 0) & (loc < N_HALF)
                cref[pl.ds(k * 16, 16)] = jnp.where(ok, loc * R_K + t16,
                                                    CT_TRASH)
                return 0
            lax.fori_loop(0, 8, cbody, 0)
            pltpu.sync_copy(ones_b, counts_sh.at[cref], add=True)
        return 0
    lax.fori_loop(0, EPT_P1 // 512, super_body, 0)

    plsc.subcore_barrier()
    pltpu.sync_copy(counts_sh.at[pl.ds(sid * 6400, 6400)],
                    zbuf.at[pl.ds(0, 6400)])

    def inv_body(i, _):
        v = zbuf[pl.ds(i * 16, 16)]
        zbuf[pl.ds(i * 16, 16)] = 1.0 / jnp.maximum(v, 1.0)
        return 0
    lax.fori_loop(0, 400, inv_body, 0)
    pltpu.sync_copy(zbuf.at[pl.ds(0, 6400)],
                    inv_hbm.at[pl.ds(cid * CT_HALF + sid * 6400, 6400)])


@functools.partial(
    pl.kernel, mesh=_MESH, compiler_params=_SC_PARAMS,
    out_type=jax.ShapeDtypeStruct((INV_SZ,), jnp.float32),
    scratch_types=[
        pltpu.VMEM_SHARED((CT_HALF,), jnp.float32),
        pltpu.VMEM((6400,), jnp.float32),
        pltpu.VMEM((128,), jnp.float32),
        pltpu.VMEM((512,), jnp.int32),
        pltpu.VMEM((512,), jnp.int32),
        pltpu.VMEM((128,), jnp.int32),
        pltpu.VMEM((128,), jnp.int32),
        pltpu.VMEM((128,), jnp.int32),
        pltpu.VMEM((128,), jnp.int32),
    ],
)
def _sc_prep1(dst_hbm, t_hbm, inv_hbm, counts_sh, zbuf, ones_b, dbuf, tbuf,
              c0, c1, c2, c3):
    _prep1_body(dst_hbm, t_hbm, inv_hbm, counts_sh, zbuf, ones_b, dbuf, tbuf,
                c0, c1, c2, c3)


@functools.partial(
    pl.kernel, mesh=_MESH, compiler_params=_SC_PARAMS,
    out_type=[
        jax.ShapeDtypeStruct((E_PAD,), jnp.int32),
        jax.ShapeDtypeStruct((E_PAD,), jnp.float32),
        jax.ShapeDtypeStruct((2 * E_PAD,), jnp.int32),
    ],
    scratch_types=[
        pltpu.VMEM((512,), jnp.int32),
        pltpu.VMEM((512,), jnp.int32),
        pltpu.VMEM((512,), jnp.int32),
        pltpu.VMEM((512,), jnp.int32),
        pltpu.VMEM((512,), jnp.int32),
        pltpu.VMEM((512,), jnp.float32),
        pltpu.VMEM((512,), jnp.int32),
        pltpu.VMEM((512,), jnp.int32),
        pltpu.SemaphoreType.DMA,
    ],
)
def _sc_prep2(src_hbm, dst_hbm, t_hbm, inv_hbm, gidx_hbm, s_hbm, dl_hbm,
              bsrc, bdst, bt, gflat, cflat, sflat, dl0f, dl1f, sem):
    cid = lax.axis_index("c")
    sid = lax.axis_index("s")
    wid = cid * 16 + sid

    def chunk_body(ch, _):
        base = wid * EPW_P2 + ch * 512
        pltpu.sync_copy(src_hbm.at[pl.ds(base, 512)], bsrc)
        pltpu.sync_copy(dst_hbm.at[pl.ds(base, 512)], bdst)
        pltpu.sync_copy(t_hbm.at[pl.ds(base, 512)], bt)

        def cbody(i, _c):
            sl = pl.ds(i * 16, 16)
            s16 = bsrc[sl]
            d16 = bdst[sl]
            t16 = bt[sl]
            gflat[sl] = t16 * N_K + s16
            upper = d16 >= N_HALF
            loc = d16 - jnp.where(upper, N_HALF, 0)
            cflat[sl] = jnp.where(upper, CT_HALF, 0) + loc * R_K + t16
            dl0f[sl] = jnp.where(d16 < N_HALF, d16, TRASH_ROW)
            loc1 = d16 - N_HALF
            ok1 = (loc1 >= 0) & (loc1 < N_HALF)
            dl1f[sl] = jnp.where(ok1, loc1, TRASH_ROW)
            return 0
        lax.fori_loop(0, 32, cbody, 0)

        pltpu.sync_copy(gflat, gidx_hbm.at[pl.ds(base, 512)])
        pltpu.sync_copy(dl0f, dl_hbm.at[pl.ds(base, 512)])
        pltpu.sync_copy(dl1f, dl_hbm.at[pl.ds(E_PAD + base, 512)])
        handles = [
            pltpu.async_copy(inv_hbm.at[cflat.at[pl.ds(j * 128, 128)]],
                             sflat.at[pl.ds(j * 128, 128)], sem)
            for j in range(4)
        ]
        for h in handles:
            h.wait()
        pltpu.sync_copy(sflat, s_hbm.at[pl.ds(base, 512)])
        return 0
    lax.fori_loop(0, EPW_P2 // 512, chunk_body, 0)


@functools.partial(
    pl.kernel, mesh=_MESH, compiler_params=_SC_PARAMS,
    out_type=jax.ShapeDtypeStruct((N_K, H_K), jnp.float32),
    scratch_types=[
        pltpu.VMEM_SHARED((ACC_ROWS, H_K), jnp.float32),
        pltpu.VMEM((CPB * CH,), jnp.int32),
        pltpu.VMEM((CPB * CH,), jnp.float32),
        pltpu.VMEM((CPB * CH,), jnp.int32),
        pltpu.VMEM((CH, H_K), jnp.float32),
        pltpu.VMEM((CH, H_K), jnp.float32),
        pltpu.SemaphoreType.DMA,
    ],
)
def _sc_layer(root_hbm, y_hbm, gidx_hbm, s_hbm, dl_hbm, out_hbm,
              acc_sh, gsup, ssup, dsup, rows0, rows1, sem_g):
    cid = lax.axis_index("c")
    sid = lax.axis_index("s")
    nbase = cid * N_HALF
    r0 = sid * 1568
    SUP = CPB * CH

    @pl.when(sid < 15)
    def _init_main():
        pltpu.sync_copy(root_hbm.at[pl.ds(nbase + r0, 1568)],
                        acc_sh.at[pl.ds(r0, 1568)])

    @pl.when(sid == 15)
    def _init_tail():
        pltpu.sync_copy(root_hbm.at[pl.ds(nbase + 23520, 1480)],
                        acc_sh.at[pl.ds(23520, 1480)])

    plsc.subcore_barrier()

    iota16 = lax.iota(jnp.int32, 16)
    ebase = sid * EPT_P1
    dlbase = cid * E_PAD + ebase

    def scale(off, rref):
        def sb(i, _):
            for k in range(16):
                e = i * 16 + k
                ridx = jnp.full((16,), e, jnp.int32)
                sk = plsc.load_gather(ssup, [jnp.full((16,), off + e,
                                                      jnp.int32)])
                for p in range(4):
                    cidx = iota16 + (p * 16)
                    v = plsc.load_gather(rref, [ridx, cidx])
                    plsc.store_scatter(rref, [ridx, cidx], v * sk)
            return 0
        lax.fori_loop(0, CH // 16, sb, 0)

    def super_body(sc, _):
        b = sc * SUP
        pltpu.sync_copy(gidx_hbm.at[pl.ds(ebase + b, SUP)], gsup)
        pltpu.sync_copy(s_hbm.at[pl.ds(ebase + b, SUP)], ssup)
        pltpu.sync_copy(dl_hbm.at[pl.ds(dlbase + b, SUP)], dsup)
        pltpu.async_copy(y_hbm.at[gsup.at[pl.ds(0, CH)]], rows0, sem_g)
        for j in range(CPB):
            rref = rows0 if j % 2 == 0 else rows1
            nref = rows1 if j % 2 == 0 else rows0
            if j + 1 < CPB:
                pltpu.async_copy(y_hbm.at[gsup.at[pl.ds((j + 1) * CH, CH)]],
                                 nref, sem_g)
            pltpu.make_async_copy(y_hbm.at[pl.ds(0, CH)], rref, sem_g).wait()
            scale(j * CH, rref)
            pltpu.sync_copy(rref, acc_sh.at[dsup.at[pl.ds(j * CH, CH)]],
                            add=True)
        return 0
    lax.fori_loop(0, EPT_P1 // SUP, super_body, 0)

    plsc.subcore_barrier()

    @pl.when(sid < 15)
    def _out_main():
        pltpu.sync_copy(acc_sh.at[pl.ds(r0, 1568)],
                        out_hbm.at[pl.ds(nbase + r0, 1568)])

    @pl.when(sid == 15)
    def _out_tail():
        pltpu.sync_copy(acc_sh.at[pl.ds(23520, 1480)],
                        out_hbm.at[pl.ds(nbase + 23520, 1480)])


def kernel(x_user, W_user, b_user, item_emb, W_rel, W_root, bias, edge_index, edge_type):
    h_user = _user_matmul(x_user, W_user, b_user)
    x = jnp.concatenate([h_user, item_emb], axis=0)

    src = edge_index[0]
    dst = edge_index[1]
    pad = E_PAD - E_K
    src_p = jnp.concatenate([src, jnp.zeros((pad,), jnp.int32)])
    dst_p = jnp.concatenate([dst, jnp.full((pad,), PAD_DST, jnp.int32)])
    t_p = jnp.concatenate([edge_type, jnp.zeros((pad,), jnp.int32)])

    inv = _sc_prep1(dst_p, t_p)
    gidx, s_e, dl = _sc_prep2(src_p, dst_p, t_p, inv)

    for l in range(L_K):
        root, y = _layer_matmul(x, W_root[l], W_rel[l], bias[l], relu=(l > 0))
        x = _sc_layer(root, y.reshape(R_K * N_K, H_K), gidx, s_e, dl)
    return x


# speedup vs baseline: 9.3357x; 1.6707x over previous
import functools
import jax
import jax.numpy as jnp
from jax import lax
from jax.experimental import pallas as pl
from jax.experimental.pallas import tpu as pltpu
from jax.experimental.pallas import tpu_sc as plsc

N_K = 50000
H_K = 64
R_K = 4
L_K = 3
E_K = 800000
E_PAD = 802816
PAD_DST = 50001

N_HALF = 25000
TRASH_ROW = 25000
ACC_ROWS = 25008
CH = 112
CPB = 4
EB = CH * H_K * 4

CT_HALF = 102400
CT_TRASH = 100096
INV_SZ = 2 * CT_HALF

EPT_P1 = 50176
EPW_P2 = 25088

_MESH = plsc.VectorSubcoreMesh(core_axis_name="c", subcore_axis_name="s")
_SC_PARAMS = pltpu.CompilerParams(needs_layout_passes=False,
                                  use_tc_tiling_on_sc=False)


def _user_mm_body(x_ref, w_ref, b_ref, o_ref):
    o_ref[...] = jnp.dot(x_ref[...], w_ref[...],
                         preferred_element_type=jnp.float32) + b_ref[...]


def _user_matmul(x_user, W_user, b_user):
    M, K = x_user.shape
    H = W_user.shape[1]
    BM = 1000
    return pl.pallas_call(
        _user_mm_body,
        grid=(M // BM,),
        in_specs=[
            pl.BlockSpec((BM, K), lambda i: (i, 0)),
            pl.BlockSpec((K, H), lambda i: (0, 0)),
            pl.BlockSpec((1, H), lambda i: (0, 0)),
        ],
        out_specs=pl.BlockSpec((BM, H), lambda i: (i, 0)),
        out_shape=jax.ShapeDtypeStruct((M, H), jnp.float32),
    )(x_user, W_user, b_user.reshape(1, H))


def _layer_mm_body(relu, x_ref, wroot_ref, wrel_ref, bias_ref, root_ref, y_ref):
    x = x_ref[...]
    if relu:
        x = jnp.maximum(x, 0.0)
    root_ref[...] = jnp.dot(x, wroot_ref[...],
                            preferred_element_type=jnp.float32) + bias_ref[...]
    for r in range(R_K):
        y_ref[r] = jnp.dot(x, wrel_ref[r],
                           preferred_element_type=jnp.float32)


def _layer_matmul(x, W_root_l, W_rel_l, bias_l, relu):
    N, H = x.shape
    BM = 1000
    return pl.pallas_call(
        functools.partial(_layer_mm_body, relu),
        grid=(N // BM,),
        in_specs=[
            pl.BlockSpec((BM, H), lambda i: (i, 0)),
            pl.BlockSpec((H, H), lambda i: (0, 0)),
            pl.BlockSpec((R_K, H, H), lambda i: (0, 0, 0)),
            pl.BlockSpec((1, H), lambda i: (0, 0)),
        ],
        out_specs=[
            pl.BlockSpec((BM, H), lambda i: (i, 0)),
            pl.BlockSpec((R_K, BM, H), lambda i: (0, i, 0)),
        ],
        out_shape=[
            jax.ShapeDtypeStruct((N, H), jnp.float32),
            jax.ShapeDtypeStruct((R_K, N, H), jnp.float32),
        ],
    )(x, W_root_l, W_rel_l, bias_l.reshape(1, H))


def _prep1_body(dst_hbm, t_hbm, inv_hbm, counts_sh, zbuf, ones_b, dbuf, tbuf,
                c0, c1, c2, c3):
    cid = lax.axis_index("c")
    sid = lax.axis_index("s")
    crefs = [c0, c1, c2, c3]

    def zero_body(i, _):
        zbuf[pl.ds(i * 16, 16)] = jnp.zeros((16,), jnp.float32)
        return 0
    lax.fori_loop(0, 400, zero_body, 0)

    def ones_body(i, _):
        ones_b[pl.ds(i * 16, 16)] = jnp.ones((16,), jnp.float32)
        return 0
    lax.fori_loop(0, 8, ones_body, 0)

    pltpu.sync_copy(zbuf.at[pl.ds(0, 6400)],
                    counts_sh.at[pl.ds(sid * 6400, 6400)])
    plsc.subcore_barrier()

    nbase = cid * N_HALF

    def super_body(sup, _):
        base = sid * EPT_P1 + sup * 512
        pltpu.sync_copy(dst_hbm.at[pl.ds(base, 512)], dbuf)
        pltpu.sync_copy(t_hbm.at[pl.ds(base, 512)], tbuf)
        for j in range(4):
            cref = crefs[j]

            def cbody(k, _c):
                off = j * 128 + k * 16
                d16 = dbuf[pl.ds(off, 16)]
                t16 = tbuf[pl.ds(off, 16)]
                loc = d16 - nbase
                ok = (loc >= 0) & (loc < N_HALF)
                cref[pl.ds(k * 16, 16)] = jnp.where(ok, loc * R_K + t16,
                                                    CT_TRASH)
                return 0
            lax.fori_loop(0, 8, cbody, 0)
            pltpu.sync_copy(ones_b, counts_sh.at[cref], add=True)
        return 0
    lax.fori_loop(0, EPT_P1 // 512, super_body, 0)

    plsc.subcore_barrier()
    pltpu.sync_copy(counts_sh.at[pl.ds(sid * 6400, 6400)],
                    zbuf.at[pl.ds(0, 6400)])

    def inv_body(i, _):
        v = zbuf[pl.ds(i * 16, 16)]
        zbuf[pl.ds(i * 16, 16)] = 1.0 / jnp.maximum(v, 1.0)
        return 0
    lax.fori_loop(0, 400, inv_body, 0)
    pltpu.sync_copy(zbuf.at[pl.ds(0, 6400)],
                    inv_hbm.at[pl.ds(cid * CT_HALF + sid * 6400, 6400)])


@functools.partial(
    pl.kernel, mesh=_MESH, compiler_params=_SC_PARAMS,
    out_type=jax.ShapeDtypeStruct((INV_SZ,), jnp.float32),
    scratch_types=[
        pltpu.VMEM_SHARED((CT_HALF,), jnp.float32),
        pltpu.VMEM((6400,), jnp.float32),
        pltpu.VMEM((128,), jnp.float32),
        pltpu.VMEM((512,), jnp.int32),
        pltpu.VMEM((512,), jnp.int32),
        pltpu.VMEM((128,), jnp.int32),
        pltpu.VMEM((128,), jnp.int32),
        pltpu.VMEM((128,), jnp.int32),
        pltpu.VMEM((128,), jnp.int32),
    ],
)
def _sc_prep1(dst_hbm, t_hbm, inv_hbm, counts_sh, zbuf, ones_b, dbuf, tbuf,
              c0, c1, c2, c3):
    _prep1_body(dst_hbm, t_hbm, inv_hbm, counts_sh, zbuf, ones_b, dbuf, tbuf,
                c0, c1, c2, c3)


@functools.partial(
    pl.kernel, mesh=_MESH, compiler_params=_SC_PARAMS,
    out_type=[
        jax.ShapeDtypeStruct((E_PAD,), jnp.int32),
        jax.ShapeDtypeStruct((E_PAD,), jnp.float32),
        jax.ShapeDtypeStruct((2 * E_PAD,), jnp.int32),
    ],
    scratch_types=[
        pltpu.VMEM((512,), jnp.int32),
        pltpu.VMEM((512,), jnp.int32),
        pltpu.VMEM((512,), jnp.int32),
        pltpu.VMEM((512,), jnp.int32),
        pltpu.VMEM((512,), jnp.int32),
        pltpu.VMEM((512,), jnp.float32),
        pltpu.VMEM((512,), jnp.int32),
        pltpu.VMEM((512,), jnp.int32),
        pltpu.SemaphoreType.DMA,
    ],
)
def _sc_prep2(src_hbm, dst_hbm, t_hbm, inv_hbm, gidx_hbm, s_hbm, dl_hbm,
              bsrc, bdst, bt, gflat, cflat, sflat, dl0f, dl1f, sem):
    cid = lax.axis_index("c")
    sid = lax.axis_index("s")
    wid = cid * 16 + sid

    def chunk_body(ch, _):
        base = wid * EPW_P2 + ch * 512
        pltpu.sync_copy(src_hbm.at[pl.ds(base, 512)], bsrc)
        pltpu.sync_copy(dst_hbm.at[pl.ds(base, 512)], bdst)
        pltpu.sync_copy(t_hbm.at[pl.ds(base, 512)], bt)

        def cbody(i, _c):
            sl = pl.ds(i * 16, 16)
            s16 = bsrc[sl]
            d16 = bdst[sl]
            t16 = bt[sl]
            gflat[sl] = t16 * N_K + s16
            upper = d16 >= N_HALF
            loc = d16 - jnp.where(upper, N_HALF, 0)
            cflat[sl] = jnp.where(upper, CT_HALF, 0) + loc * R_K + t16
            dl0f[sl] = jnp.where(d16 < N_HALF, d16, TRASH_ROW)
            loc1 = d16 - N_HALF
            ok1 = (loc1 >= 0) & (loc1 < N_HALF)
            dl1f[sl] = jnp.where(ok1, loc1, TRASH_ROW)
            return 0
        lax.fori_loop(0, 32, cbody, 0)

        pltpu.sync_copy(gflat, gidx_hbm.at[pl.ds(base, 512)])
        pltpu.sync_copy(dl0f, dl_hbm.at[pl.ds(base, 512)])
        pltpu.sync_copy(dl1f, dl_hbm.at[pl.ds(E_PAD + base, 512)])
        handles = [
            pltpu.async_copy(inv_hbm.at[cflat.at[pl.ds(j * 128, 128)]],
                             sflat.at[pl.ds(j * 128, 128)], sem)
            for j in range(4)
        ]
        for h in handles:
            h.wait()
        pltpu.sync_copy(sflat, s_hbm.at[pl.ds(base, 512)])
        return 0
    lax.fori_loop(0, EPW_P2 // 512, chunk_body, 0)


@functools.partial(
    pl.kernel, mesh=_MESH, compiler_params=_SC_PARAMS,
    out_type=jax.ShapeDtypeStruct((N_K, H_K), jnp.float32),
    scratch_types=[
        pltpu.VMEM_SHARED((ACC_ROWS, H_K), jnp.float32),
        pltpu.VMEM((CPB * CH,), jnp.int32),
        pltpu.VMEM((CPB * CH,), jnp.int32),
        pltpu.VMEM((CPB * CH,), jnp.float32),
        pltpu.VMEM((CPB * CH,), jnp.float32),
        pltpu.VMEM((CPB * CH,), jnp.int32),
        pltpu.VMEM((CPB * CH,), jnp.int32),
        pltpu.VMEM((CH, H_K), jnp.float32),
        pltpu.VMEM((CH, H_K), jnp.float32),
        pltpu.SemaphoreType.DMA,
        pltpu.SemaphoreType.DMA,
        pltpu.SemaphoreType.DMA,
    ],
)
def _sc_layer(root_hbm, y_hbm, gidx_hbm, s_hbm, dl_hbm, out_hbm,
              acc_sh, gsup0, gsup1, ssup0, ssup1, dsup0, dsup1,
              rows0, rows1, sem_g, sem_s, sem_i):
    cid = lax.axis_index("c")
    sid = lax.axis_index("s")
    nbase = cid * N_HALF
    r0 = sid * 1568
    SUP = CPB * CH
    NSUP = EPT_P1 // SUP

    @pl.when(sid < 15)
    def _init_main():
        pltpu.sync_copy(root_hbm.at[pl.ds(nbase + r0, 1568)],
                        acc_sh.at[pl.ds(r0, 1568)])

    @pl.when(sid == 15)
    def _init_tail():
        pltpu.sync_copy(root_hbm.at[pl.ds(nbase + 23520, 1480)],
                        acc_sh.at[pl.ds(23520, 1480)])

    plsc.subcore_barrier()

    ebase = sid * EPT_P1
    dlbase = cid * E_PAD + ebase
    gref = [gsup0, gsup1]
    sref = [ssup0, ssup1]
    dref = [dsup0, dsup1]

    def load_idx(b, w):
        eb = ebase + b * SUP
        pltpu.async_copy(gidx_hbm.at[pl.ds(eb, SUP)], gref[w], sem_i)
        pltpu.async_copy(s_hbm.at[pl.ds(eb, SUP)], sref[w], sem_i)
        pltpu.async_copy(dl_hbm.at[pl.ds(dlbase + b * SUP, SUP)],
                         dref[w], sem_i)

    def wait_idx(w):
        pltpu.make_async_copy(gidx_hbm.at[pl.ds(0, SUP)], gref[w],
                              sem_i).wait()
        pltpu.make_async_copy(s_hbm.at[pl.ds(0, SUP)], sref[w],
                              sem_i).wait()
        pltpu.make_async_copy(dl_hbm.at[pl.ds(0, SUP)], dref[w],
                              sem_i).wait()

    def scale(ssup, off, rref):
        def sb(i, _):
            for k in range(4):
                e = i * 4 + k
                sk = plsc.load_gather(ssup, [jnp.full((16,), off + e,
                                                      jnp.int32)])
                for p in range(4):
                    sl = pl.ds(p * 16, 16)
                    rref[e, sl] = rref[e, sl] * sk
            return 0
        lax.fori_loop(0, CH // 4, sb, 0)

    def super_body(sup, w):
        gsup, ssup, dsup = gref[w], sref[w], dref[w]
        wait_idx(w)
        pltpu.async_copy(y_hbm.at[gsup.at[pl.ds(0, CH)]], rows0, sem_g)
        rows = [rows0, rows1]
        for j in range(CPB):
            t = j % 2
            rref = rows[t]
            if j + 1 < CPB:
                if j >= 1:
                    pltpu.make_async_copy(rows[1 - t], acc_sh.at[pl.ds(0, CH)],
                                          sem_s).wait()
                pltpu.async_copy(y_hbm.at[gsup.at[pl.ds((j + 1) * CH, CH)]],
                                 rows[1 - t], sem_g)
            pltpu.make_async_copy(y_hbm.at[pl.ds(0, CH)], rref, sem_g).wait()
            scale(ssup, j * CH, rref)
            pltpu.async_copy(rref, acc_sh.at[dsup.at[pl.ds(j * CH, CH)]],
                             sem_s, add=True)
        pltpu.make_async_copy(rows0, acc_sh.at[pl.ds(0, CH)], sem_s).wait()
        pltpu.make_async_copy(rows1, acc_sh.at[pl.ds(0, CH)], sem_s).wait()

    load_idx(0, 0)

    def sup_loop(m, _):
        sup = 2 * m
        load_idx(sup + 1, 1)
        super_body(sup, 0)

        @pl.when(sup + 2 < NSUP)
        def _pre():
            load_idx(sup + 2, 0)
        super_body(sup + 1, 1)
        return 0
    lax.fori_loop(0, NSUP // 2, sup_loop, 0)

    plsc.subcore_barrier()

    @pl.when(sid < 15)
    def _out_main():
        pltpu.sync_copy(acc_sh.at[pl.ds(r0, 1568)],
                        out_hbm.at[pl.ds(nbase + r0, 1568)])

    @pl.when(sid == 15)
    def _out_tail():
        pltpu.sync_copy(acc_sh.at[pl.ds(23520, 1480)],
                        out_hbm.at[pl.ds(nbase + 23520, 1480)])


def kernel(x_user, W_user, b_user, item_emb, W_rel, W_root, bias, edge_index, edge_type):
    h_user = _user_matmul(x_user, W_user, b_user)
    x = jnp.concatenate([h_user, item_emb], axis=0)

    src = edge_index[0]
    dst = edge_index[1]
    pad = E_PAD - E_K
    src_p = jnp.concatenate([src, jnp.zeros((pad,), jnp.int32)])
    dst_p = jnp.concatenate([dst, jnp.full((pad,), PAD_DST, jnp.int32)])
    t_p = jnp.concatenate([edge_type, jnp.zeros((pad,), jnp.int32)])

    inv = _sc_prep1(dst_p, t_p)
    gidx, s_e, dl = _sc_prep2(src_p, dst_p, t_p, inv)

    for l in range(L_K):
        root, y = _layer_matmul(x, W_root[l], W_rel[l], bias[l], relu=(l > 0))
        x = _sc_layer(root, y.reshape(R_K * N_K, H_K), gidx, s_e, dl)
    return x
